# baseline (speedup 1.0000x reference)
import functools

import jax
import jax.numpy as jnp
import numpy as np
from jax import lax
from jax.experimental import pallas as pl
from jax.experimental.pallas import tpu as pltpu
from jax.experimental.pallas import tpu_sc as plsc

N_NODES = 10000
N_EDGES = 320000
HID = 128
N_OUT = 64
N_BATCH = 64
D_DRONE = 51
N_LAYERS = 3
LANES = 16

NC = 2
NS = 16
NW = NC * NS
EPW = N_EDGES // NW
CHUNK = 80
NCHUNK = EPW // CHUNK
CPR = 400
NCP = N_NODES // CPR

BLK = 400
GRID = N_NODES // BLK
F32 = jnp.float32

_SC_MESH = plsc.VectorSubcoreMesh(core_axis_name="c", subcore_axis_name="s")


BF16 = jnp.bfloat16
assert (NCHUNK - 1) % 4 == 0


def _edge_chunk_loop(wid, a_hbm, c_hbm, src_hbm, dst_hbm,
                     srcs, dsts, avs, cvs, rvs, s_sh, gsems, ssems):
    def load_idx(t, p, d):
        off = wid * EPW + t * CHUNK
        pltpu.sync_copy(src_hbm.at[pl.ds(off, CHUNK)], srcs[p])
        pltpu.sync_copy(dst_hbm.at[pl.ds(off, CHUNK)], dsts[d])

    def issue_gathers(p, d):
        pltpu.async_copy(a_hbm.at[srcs[p]], avs[p], gsems[p])
        pltpu.async_copy(c_hbm.at[dsts[d]], cvs[p], gsems[p])

    def wait_gathers(p, d):
        pltpu.make_async_copy(a_hbm.at[srcs[p]], avs[p], gsems[p]).wait()
        pltpu.make_async_copy(c_hbm.at[dsts[d]], cvs[p], gsems[p]).wait()

    def issue_scatter(p, d):
        pltpu.async_copy(rvs[p], s_sh.at[dsts[d]], ssems[p], add=True)

    def wait_scatter(p, d):
        pltpu.make_async_copy(rvs[p], s_sh.at[dsts[d]], ssems[p]).wait()

    def compute(p):
        ap = avs[p]
        cp = cvs[p]
        rp = rvs[p]

        msk = jnp.int32(-65536)

        def row_body(i, cc):
            for k in range(HID // 32):
                aw = ap[i, pl.ds(LANES * k, LANES)]
                cw = cp[i, pl.ds(LANES * k, LANES)]
                a_lo = lax.bitcast_convert_type(aw << 16, F32)
                c_lo = lax.bitcast_convert_type(cw << 16, F32)
                a_hi = lax.bitcast_convert_type(aw & msk, F32)
                c_hi = lax.bitcast_convert_type(cw & msk, F32)
                rp[i, pl.ds(32 * k, LANES)] = jnp.maximum(a_lo - c_lo, 0.0)
                rp[i, pl.ds(32 * k + LANES, LANES)] = jnp.maximum(a_hi - c_hi, 0.0)
            return cc

        lax.fori_loop(0, CHUNK, row_body, 0)

    load_idx(0, 0, 0)
    issue_gathers(0, 0)

    def outer(j, carry):
        for b in range(4):
            t = j * 4 + b
            p = b % 2
            q = 1 - p
            d = b
            dn = (b + 1) % 4

            @pl.when(t >= 2)
            def _():
                wait_scatter(p, (b + 2) % 4)

            load_idx(t + 1, q, dn)
            issue_gathers(q, dn)
            wait_gathers(p, d)
            compute(p)
            issue_scatter(p, d)
        return carry

    lax.fori_loop(0, (NCHUNK - 1) // 4, outer, 0)
    wait_scatter(0, 2)
    wait_gathers(0, 0)
    compute(0)
    issue_scatter(0, 0)
    wait_scatter(0, 0)
    wait_scatter(1, 3)


def _rows_copy(sid, pairs):
    r0 = sid * CPR
    for src, dst in pairs:
        pltpu.sync_copy(src(r0), dst(r0))

    @pl.when(sid + NS < NCP)
    def _():
        r1 = (sid + NS) * CPR
        for src, dst in pairs:
            pltpu.sync_copy(src(r1), dst(r1))


_SC_SCRATCH = (
    [pltpu.VMEM((CHUNK,), jnp.int32) for _ in range(2)]
    + [pltpu.VMEM((CHUNK,), jnp.int32) for _ in range(4)]
    + [pltpu.VMEM((CHUNK, HID // 2), jnp.int32) for _ in range(4)]
    + [pltpu.VMEM((CHUNK, HID), F32) for _ in range(2)]
    + [pltpu.VMEM_SHARED((N_NODES, HID), F32)]
    + [pltpu.SemaphoreType.DMA for _ in range(4)]
)


def _sc_unpack_bufs(bufs):
    srcs = bufs[0:2]
    dsts = bufs[2:6]
    avs = bufs[6:8]
    cvs = bufs[8:10]
    rvs = bufs[10:12]
    s_sh = bufs[12]
    gsems = bufs[13:15]
    ssems = bufs[15:17]
    return srcs, dsts, avs, cvs, rvs, s_sh, gsems, ssems


@functools.partial(
    pl.kernel,
    out_type=(
        jax.ShapeDtypeStruct((NC, N_NODES, HID), F32),
        jax.ShapeDtypeStruct((NC, N_NODES, HID), F32),
    ),
    mesh=_SC_MESH,
    scratch_types=_SC_SCRATCH,
    compiler_params=pltpu.CompilerParams(use_tc_tiling_on_sc=False),
)
def _sc_edge_cnt(a_hbm, c_hbm, src_hbm, dst_hbm, zs_hbm,
                 s_out, cnt_out, *bufs):
    srcs, dsts, avs, cvs, rvs, s_sh, gsems, ssems = _sc_unpack_bufs(bufs)
    cid = lax.axis_index("c")
    sid = lax.axis_index("s")
    wid = sid * NC + cid
    _rows_copy(sid, [
        (lambda r: zs_hbm.at[pl.ds(r, CPR)], lambda r: s_sh.at[pl.ds(r, CPR)]),
    ])
    plsc.subcore_barrier()
    _edge_chunk_loop(wid, a_hbm, c_hbm, src_hbm, dst_hbm,
                     srcs, dsts, avs, cvs, rvs, s_sh, gsems, ssems)
    plsc.subcore_barrier()
    _rows_copy(sid, [
        (lambda r: s_sh.at[pl.ds(r, CPR)],
         lambda r: s_out.at[cid, pl.ds(r, CPR)]),
    ])
    plsc.subcore_barrier()
    _rows_copy(sid, [
        (lambda r: zs_hbm.at[pl.ds(r, CPR)], lambda r: s_sh.at[pl.ds(r, CPR)]),
    ])

    ones_v = rvs[0]

    def ones_row(i, cc):
        for g in range(HID // LANES):
            ones_v[i, pl.ds(g * LANES, LANES)] = jnp.full((LANES,), 1.0, F32)
        return cc

    lax.fori_loop(0, CHUNK, ones_row, 0)
    plsc.subcore_barrier()

    def cnt_load_issue(t, p):
        off = wid * EPW + t * CHUNK
        pltpu.sync_copy(dst_hbm.at[pl.ds(off, CHUNK)], dsts[p])
        pltpu.async_copy(ones_v, s_sh.at[dsts[p]], ssems[p], add=True)

    def cnt_wait(p):
        pltpu.make_async_copy(ones_v, s_sh.at[dsts[p]], ssems[p]).wait()

    def cnt_outer(k, carry):
        for b in range(2):
            t = k * 2 + b

            @pl.when(t >= 2)
            def _():
                cnt_wait(b)

            cnt_load_issue(t, b)
        return carry

    lax.fori_loop(0, (NCHUNK - 1) // 2, cnt_outer, 0)
    cnt_wait(0)
    cnt_load_issue(NCHUNK - 1, 0)
    cnt_wait(0)
    cnt_wait(1)
    plsc.subcore_barrier()
    _rows_copy(sid, [
        (lambda r: s_sh.at[pl.ds(r, CPR)],
         lambda r: cnt_out.at[cid, pl.ds(r, CPR)]),
    ])


@functools.partial(
    pl.kernel,
    out_type=jax.ShapeDtypeStruct((NC, N_NODES, HID), F32),
    mesh=_SC_MESH,
    scratch_types=_SC_SCRATCH,
    compiler_params=pltpu.CompilerParams(use_tc_tiling_on_sc=False),
)
def _sc_edge(a_hbm, c_hbm, src_hbm, dst_hbm, zs_hbm,
             s_out, *bufs):
    srcs, dsts, avs, cvs, rvs, s_sh, gsems, ssems = _sc_unpack_bufs(bufs)
    cid = lax.axis_index("c")
    sid = lax.axis_index("s")
    wid = sid * NC + cid
    _rows_copy(sid, [
        (lambda r: zs_hbm.at[pl.ds(r, CPR)], lambda r: s_sh.at[pl.ds(r, CPR)]),
    ])
    plsc.subcore_barrier()
    _edge_chunk_loop(wid, a_hbm, c_hbm, src_hbm, dst_hbm,
                     srcs, dsts, avs, cvs, rvs, s_sh, gsems, ssems)
    plsc.subcore_barrier()
    _rows_copy(sid, [
        (lambda r: s_sh.at[pl.ds(r, CPR)],
         lambda r: s_out.at[cid, pl.ds(r, CPR)]),
    ])


def _dot(a, b):
    return jnp.dot(a, b, preferred_element_type=F32)


def _pre_body(x, b2d, nwt, nb, dfp, dwt, db, wat, b1, wbt,
              h_out, a_out, c_out):
    demb = _dot(dfp[...], dwt[...]) + db[...]
    h = _dot(x[...], nwt[...]) + nb[...]
    oneh = (b2d[...] == lax.broadcasted_iota(jnp.int32, (BLK, N_BATCH), 1)
            ).astype(F32)
    h = h + _dot(oneh, demb)
    h_out[...] = h
    a_out[...] = (_dot(h, wat[...]) + b1[...]).astype(BF16)
    c_out[...] = _dot(h, wbt[...]).astype(BF16)


def _layer_update(h, s2, c2, w2t, b2, swt, sb, owat, owbt, ob, g, bb):
    sarr = s2[...]
    s = sarr[0] + sarr[1]
    carr = c2[...]
    cnt = carr[0, :, 0:1] + carr[1, :, 0:1]
    aggr = (_dot(s, w2t[...]) + cnt * b2[...]) / jnp.maximum(cnt, 1.0)
    selfp = _dot(h[...], swt[...]) + sb[...]
    cc = _dot(selfp, owat[...]) + _dot(aggr, owbt[...]) + ob[...]
    mu = jnp.mean(cc, axis=-1, keepdims=True)
    var = jnp.mean((cc - mu) ** 2, axis=-1, keepdims=True)
    ln = (cc - mu) * lax.rsqrt(var + 1e-5) * g[...] + bb[...]
    return h[...] + jnp.maximum(ln, 0.0)


def _mid_body(h, s2, c2, w2t, b2, swt, sb, owat, owbt, ob, g, bb,
              want, b1n, wbnt, h_out, a_out, c_out):
    hn = _layer_update(h, s2, c2, w2t, b2, swt, sb, owat, owbt, ob, g, bb)
    h_out[...] = hn
    a_out[...] = (_dot(hn, want[...]) + b1n[...]).astype(BF16)
    c_out[...] = _dot(hn, wbnt[...]).astype(BF16)


def _post_body(h, s2, c2, w2t, b2, swt, sb, owat, owbt, ob, g, bb,
               pjt, pjb, y_out):
    hn = _layer_update(h, s2, c2, w2t, b2, swt, sb, owat, owbt, ob, g, bb)
    y_out[...] = _dot(hn, pjt[...]) + pjb[...]


def _rows_spec(ncol):
    return pl.BlockSpec((BLK, ncol), lambda i: (i, 0))


def _full_spec(shape):
    nd = len(shape)
    return pl.BlockSpec(shape, lambda i, _nd=nd: (0,) * _nd)


def _part_spec(ncol):
    return pl.BlockSpec((NC, BLK, ncol), lambda i: (0, i, 0))


def kernel(x, edge_index, drone_feat, batch, node_w, node_b, drone_w, drone_b,
           edge_w1, edge_b1, edge_w2, edge_b2, self_w, self_b, out_w, out_b,
           ln_g, ln_b, proj_w, proj_b):
    src = edge_index[0]
    dst = edge_index[1]
    b2d = batch.reshape(N_NODES, 1)

    nwt = node_w.T
    nb = node_b.reshape(1, HID)
    dfp = jnp.pad(drone_feat, ((0, 0), (0, N_BATCH - D_DRONE)))
    dwt = jnp.pad(drone_w.T, ((0, N_BATCH - D_DRONE), (0, 0)))
    db = drone_b.reshape(1, HID)

    wat = [(edge_w1[i, :, :HID] + edge_w1[i, :, HID:]).T for i in range(N_LAYERS)]
    wbt = [edge_w1[i, :, HID:].T for i in range(N_LAYERS)]
    b1 = [edge_b1[i].reshape(1, HID) for i in range(N_LAYERS)]
    perm = np.concatenate([
        np.concatenate([np.arange(32 * k, 32 * k + 32, 2),
                        np.arange(32 * k + 1, 32 * k + 32, 2)])
        for k in range(HID // 32)
    ])
    w2t = [edge_w2[i].T[perm, :] for i in range(N_LAYERS)]
    b2 = [edge_b2[i].reshape(1, HID) for i in range(N_LAYERS)]
    swt = [self_w[i].T for i in range(N_LAYERS)]
    sb = [self_b[i].reshape(1, HID) for i in range(N_LAYERS)]
    owat = [out_w[i, :, :HID].T for i in range(N_LAYERS)]
    owbt = [out_w[i, :, HID:].T for i in range(N_LAYERS)]
    ob = [out_b[i].reshape(1, HID) for i in range(N_LAYERS)]
    gs = [ln_g[i].reshape(1, HID) for i in range(N_LAYERS)]
    bbs = [ln_b[i].reshape(1, HID) for i in range(N_LAYERS)]
    pjt = proj_w.T
    pjb = proj_b.reshape(1, N_OUT)

    zs = jnp.zeros((N_NODES, HID), F32)

    nrow_shape = jax.ShapeDtypeStruct((N_NODES, HID), F32)
    nrow_bf16 = jax.ShapeDtypeStruct((N_NODES, HID), BF16)
    pre_out = [nrow_shape, nrow_bf16, nrow_bf16]
    h, a, c = pl.pallas_call(
        _pre_body,
        grid=(GRID,),
        in_specs=[
            _rows_spec(HID), pl.BlockSpec((BLK, 1), lambda i: (i, 0)),
            _full_spec((HID, HID)), _full_spec((1, HID)),
            _full_spec((N_BATCH, N_BATCH)), _full_spec((N_BATCH, HID)),
            _full_spec((1, HID)),
            _full_spec((HID, HID)), _full_spec((1, HID)),
            _full_spec((HID, HID)),
        ],
        out_specs=[_rows_spec(HID)] * 3,
        out_shape=pre_out,
    )(x, b2d, nwt, nb, dfp, dwt, db, wat[0], b1[0], wbt[0])

    def _pack32(arr):
        return lax.bitcast_convert_type(
            arr.reshape(N_NODES, HID // 2, 2), jnp.int32)

    s2, cnt_full = _sc_edge_cnt(_pack32(a), _pack32(c), src, dst, zs)
    c2 = cnt_full[:, :, :8]

    mid_in_specs = [
        _rows_spec(HID), _part_spec(HID), _part_spec(8),
        _full_spec((HID, HID)), _full_spec((1, HID)),
        _full_spec((HID, HID)), _full_spec((1, HID)),
        _full_spec((HID, HID)), _full_spec((HID, HID)), _full_spec((1, HID)),
        _full_spec((1, HID)), _full_spec((1, HID)),
    ]

    for i in range(N_LAYERS - 1):
        h, a, c = pl.pallas_call(
            _mid_body,
            grid=(GRID,),
            in_specs=mid_in_specs + [
                _full_spec((HID, HID)), _full_spec((1, HID)),
                _full_spec((HID, HID)),
            ],
            out_specs=[_rows_spec(HID)] * 3,
            out_shape=pre_out,
        )(h, s2, c2, w2t[i], b2[i], swt[i], sb[i], owat[i], owbt[i], ob[i],
          gs[i], bbs[i], wat[i + 1], b1[i + 1], wbt[i + 1])
        s2 = _sc_edge(_pack32(a), _pack32(c), src, dst, zs)

    y = pl.pallas_call(
        _post_body,
        grid=(GRID,),
        in_specs=mid_in_specs + [
            _full_spec((HID, N_OUT)), _full_spec((1, N_OUT)),
        ],
        out_specs=_rows_spec(N_OUT),
        out_shape=jax.ShapeDtypeStruct((N_NODES, N_OUT), F32),
    )(h, s2, c2, w2t[2], b2[2], swt[2], sb[2], owat[2], owbt[2], ob[2],
      gs[2], bbs[2], pjt, pjb)
    return y


# speedup vs baseline: 1.5378x; 1.5378x over previous
import functools

import jax
import jax.numpy as jnp
import numpy as np
from jax import lax
from jax.experimental import pallas as pl
from jax.experimental.pallas import tpu as pltpu
from jax.experimental.pallas import tpu_sc as plsc

N_NODES = 10000
N_EDGES = 320000
HID = 128
N_OUT = 64
N_BATCH = 64
D_DRONE = 51
N_LAYERS = 3
LANES = 16

NC = 2
NS = 16
NW = NC * NS
EPW = N_EDGES // NW
CHUNK = 80
NCHUNK = EPW // CHUNK
CPR = 400
NCP = N_NODES // CPR

BLK = 400
GRID = N_NODES // BLK
F32 = jnp.float32

_SC_MESH = plsc.VectorSubcoreMesh(core_axis_name="c", subcore_axis_name="s")


BF16 = jnp.bfloat16
assert (NCHUNK - 1) % 4 == 0


def _edge_chunk_loop(wid, a_hbm, c_hbm, src_hbm, dst_hbm,
                     srcs, dsts, avs, cvs, s_sh, gsems, ssems):
    def load_idx(t, p):
        off = wid * EPW + t * CHUNK
        pltpu.sync_copy(src_hbm.at[pl.ds(off, CHUNK)], srcs[p])
        pltpu.sync_copy(dst_hbm.at[pl.ds(off, CHUNK)], dsts[p])

    def issue_gathers(p):
        pltpu.async_copy(a_hbm.at[srcs[p]], avs[p], gsems[p])
        pltpu.async_copy(c_hbm.at[dsts[p]], cvs[p], gsems[p])

    def wait_gathers(p):
        pltpu.make_async_copy(a_hbm.at[srcs[p]], avs[p], gsems[p]).wait()
        pltpu.make_async_copy(c_hbm.at[dsts[p]], cvs[p], gsems[p]).wait()

    def issue_scatter(p):
        pltpu.async_copy(avs[p], s_sh.at[dsts[p]], ssems[p], add=True)

    def wait_scatter(p):
        pltpu.make_async_copy(avs[p], s_sh.at[dsts[p]], ssems[p]).wait()

    def compute(p):
        ap = avs[p]
        cp = cvs[p]

        def row_body(i2, cc):
            for u in range(2):
                i = i2 * 2 + u
                for g in range(HID // LANES):
                    sl = pl.ds(g * LANES, LANES)
                    ap[i, sl] = jnp.maximum(ap[i, sl] - cp[i, sl], 0.0)
            return cc

        lax.fori_loop(0, CHUNK // 2, row_body, 0)

    load_idx(0, 0)
    issue_gathers(0)

    def outer(j, carry):
        for b in range(4):
            t = j * 4 + b
            p = b % 2
            q = 1 - p

            @pl.when(t >= 1)
            def _():
                wait_scatter(q)

            load_idx(t + 1, q)
            issue_gathers(q)
            wait_gathers(p)
            compute(p)
            issue_scatter(p)
        return carry

    lax.fori_loop(0, (NCHUNK - 1) // 4, outer, 0)
    wait_scatter(1)
    wait_gathers(0)
    compute(0)
    issue_scatter(0)
    wait_scatter(0)


def _rows_copy(sid, pairs):
    r0 = sid * CPR
    for src, dst in pairs:
        pltpu.sync_copy(src(r0), dst(r0))

    @pl.when(sid + NS < NCP)
    def _():
        r1 = (sid + NS) * CPR
        for src, dst in pairs:
            pltpu.sync_copy(src(r1), dst(r1))


_SC_SCRATCH = (
    [pltpu.VMEM((CHUNK,), jnp.int32) for _ in range(4)]
    + [pltpu.VMEM((CHUNK, HID), F32) for _ in range(4)]
    + [pltpu.VMEM_SHARED((N_NODES, HID), F32)]
    + [pltpu.SemaphoreType.DMA for _ in range(4)]
)


def _sc_unpack_bufs(bufs):
    srcs = bufs[0:2]
    dsts = bufs[2:4]
    avs = bufs[4:6]
    cvs = bufs[6:8]
    s_sh = bufs[8]
    gsems = bufs[9:11]
    ssems = bufs[11:13]
    return srcs, dsts, avs, cvs, s_sh, gsems, ssems


@functools.partial(
    pl.kernel,
    out_type=(
        jax.ShapeDtypeStruct((NC, N_NODES, HID), F32),
        jax.ShapeDtypeStruct((NC, N_NODES, HID), F32),
    ),
    mesh=_SC_MESH,
    scratch_types=_SC_SCRATCH,
)
def _sc_edge_cnt(a_hbm, c_hbm, src_hbm, dst_hbm, zs_hbm,
                 s_out, cnt_out, *bufs):
    srcs, dsts, avs, cvs, s_sh, gsems, ssems = _sc_unpack_bufs(bufs)
    cid = lax.axis_index("c")
    sid = lax.axis_index("s")
    wid = sid * NC + cid
    _rows_copy(sid, [
        (lambda r: zs_hbm.at[pl.ds(r, CPR)], lambda r: s_sh.at[pl.ds(r, CPR)]),
    ])
    plsc.subcore_barrier()
    _edge_chunk_loop(wid, a_hbm, c_hbm, src_hbm, dst_hbm,
                     srcs, dsts, avs, cvs, s_sh, gsems, ssems)
    plsc.subcore_barrier()
    _rows_copy(sid, [
        (lambda r: s_sh.at[pl.ds(r, CPR)],
         lambda r: s_out.at[cid, pl.ds(r, CPR)]),
    ])
    plsc.subcore_barrier()
    _rows_copy(sid, [
        (lambda r: zs_hbm.at[pl.ds(r, CPR)], lambda r: s_sh.at[pl.ds(r, CPR)]),
    ])

    ones_v = cvs[0]

    def ones_row(i, cc):
        for g in range(HID // LANES):
            ones_v[i, pl.ds(g * LANES, LANES)] = jnp.full((LANES,), 1.0, F32)
        return cc

    lax.fori_loop(0, CHUNK, ones_row, 0)
    plsc.subcore_barrier()

    def cnt_load_issue(t, p):
        off = wid * EPW + t * CHUNK
        pltpu.sync_copy(dst_hbm.at[pl.ds(off, CHUNK)], dsts[p])
        pltpu.async_copy(ones_v, s_sh.at[dsts[p]], ssems[p], add=True)

    def cnt_wait(p):
        pltpu.make_async_copy(ones_v, s_sh.at[dsts[p]], ssems[p]).wait()

    def cnt_outer(k, carry):
        for b in range(2):
            t = k * 2 + b

            @pl.when(t >= 2)
            def _():
                cnt_wait(b)

            cnt_load_issue(t, b)
        return carry

    lax.fori_loop(0, (NCHUNK - 1) // 2, cnt_outer, 0)
    cnt_wait(0)
    cnt_load_issue(NCHUNK - 1, 0)
    cnt_wait(0)
    cnt_wait(1)
    plsc.subcore_barrier()
    _rows_copy(sid, [
        (lambda r: s_sh.at[pl.ds(r, CPR)],
         lambda r: cnt_out.at[cid, pl.ds(r, CPR)]),
    ])


@functools.partial(
    pl.kernel,
    out_type=jax.ShapeDtypeStruct((NC, N_NODES, HID), F32),
    mesh=_SC_MESH,
    scratch_types=_SC_SCRATCH,
)
def _sc_edge(a_hbm, c_hbm, src_hbm, dst_hbm, zs_hbm,
             s_out, *bufs):
    srcs, dsts, avs, cvs, s_sh, gsems, ssems = _sc_unpack_bufs(bufs)
    cid = lax.axis_index("c")
    sid = lax.axis_index("s")
    wid = sid * NC + cid
    _rows_copy(sid, [
        (lambda r: zs_hbm.at[pl.ds(r, CPR)], lambda r: s_sh.at[pl.ds(r, CPR)]),
    ])
    plsc.subcore_barrier()
    _edge_chunk_loop(wid, a_hbm, c_hbm, src_hbm, dst_hbm,
                     srcs, dsts, avs, cvs, s_sh, gsems, ssems)
    plsc.subcore_barrier()
    _rows_copy(sid, [
        (lambda r: s_sh.at[pl.ds(r, CPR)],
         lambda r: s_out.at[cid, pl.ds(r, CPR)]),
    ])


def _dot(a, b):
    return jnp.dot(a, b, preferred_element_type=F32)


def _pre_body(x, b2d, nwt, nb, dfp, dwt, db, wat, b1, wbt,
              h_out, a_out, c_out):
    demb = _dot(dfp[...], dwt[...]) + db[...]
    h = _dot(x[...], nwt[...]) + nb[...]
    oneh = (b2d[...] == lax.broadcasted_iota(jnp.int32, (BLK, N_BATCH), 1)
            ).astype(F32)
    h = h + _dot(oneh, demb)
    h_out[...] = h
    a_out[...] = _dot(h, wat[...]) + b1[...]
    c_out[...] = _dot(h, wbt[...])


def _layer_update(h, s2, c2, w2t, b2, swt, sb, owat, owbt, ob, g, bb):
    sarr = s2[...]
    s = sarr[0] + sarr[1]
    carr = c2[...]
    cnt = carr[0, :, 0:1] + carr[1, :, 0:1]
    aggr = (_dot(s, w2t[...]) + cnt * b2[...]) / jnp.maximum(cnt, 1.0)
    selfp = _dot(h[...], swt[...]) + sb[...]
    cc = _dot(selfp, owat[...]) + _dot(aggr, owbt[...]) + ob[...]
    mu = jnp.mean(cc, axis=-1, keepdims=True)
    var = jnp.mean((cc - mu) ** 2, axis=-1, keepdims=True)
    ln = (cc - mu) * lax.rsqrt(var + 1e-5) * g[...] + bb[...]
    return h[...] + jnp.maximum(ln, 0.0)


def _mid_body(h, s2, c2, w2t, b2, swt, sb, owat, owbt, ob, g, bb,
              want, b1n, wbnt, h_out, a_out, c_out):
    hn = _layer_update(h, s2, c2, w2t, b2, swt, sb, owat, owbt, ob, g, bb)
    h_out[...] = hn
    a_out[...] = _dot(hn, want[...]) + b1n[...]
    c_out[...] = _dot(hn, wbnt[...])


def _post_body(h, s2, c2, w2t, b2, swt, sb, owat, owbt, ob, g, bb,
               pjt, pjb, y_out):
    hn = _layer_update(h, s2, c2, w2t, b2, swt, sb, owat, owbt, ob, g, bb)
    y_out[...] = _dot(hn, pjt[...]) + pjb[...]


def _rows_spec(ncol):
    return pl.BlockSpec((BLK, ncol), lambda i: (i, 0))


def _full_spec(shape):
    nd = len(shape)
    return pl.BlockSpec(shape, lambda i, _nd=nd: (0,) * _nd)


def _part_spec(ncol):
    return pl.BlockSpec((NC, BLK, ncol), lambda i: (0, i, 0))


def kernel(x, edge_index, drone_feat, batch, node_w, node_b, drone_w, drone_b,
           edge_w1, edge_b1, edge_w2, edge_b2, self_w, self_b, out_w, out_b,
           ln_g, ln_b, proj_w, proj_b):
    src = edge_index[0]
    dst = edge_index[1]
    b2d = batch.reshape(N_NODES, 1)

    nwt = node_w.T
    nb = node_b.reshape(1, HID)
    dfp = jnp.pad(drone_feat, ((0, 0), (0, N_BATCH - D_DRONE)))
    dwt = jnp.pad(drone_w.T, ((0, N_BATCH - D_DRONE), (0, 0)))
    db = drone_b.reshape(1, HID)

    wat = [(edge_w1[i, :, :HID] + edge_w1[i, :, HID:]).T for i in range(N_LAYERS)]
    wbt = [edge_w1[i, :, HID:].T for i in range(N_LAYERS)]
    b1 = [edge_b1[i].reshape(1, HID) for i in range(N_LAYERS)]
    w2t = [edge_w2[i].T for i in range(N_LAYERS)]
    b2 = [edge_b2[i].reshape(1, HID) for i in range(N_LAYERS)]
    swt = [self_w[i].T for i in range(N_LAYERS)]
    sb = [self_b[i].reshape(1, HID) for i in range(N_LAYERS)]
    owat = [out_w[i, :, :HID].T for i in range(N_LAYERS)]
    owbt = [out_w[i, :, HID:].T for i in range(N_LAYERS)]
    ob = [out_b[i].reshape(1, HID) for i in range(N_LAYERS)]
    gs = [ln_g[i].reshape(1, HID) for i in range(N_LAYERS)]
    bbs = [ln_b[i].reshape(1, HID) for i in range(N_LAYERS)]
    pjt = proj_w.T
    pjb = proj_b.reshape(1, N_OUT)

    zs = jnp.zeros((N_NODES, HID), F32)

    nrow_shape = jax.ShapeDtypeStruct((N_NODES, HID), F32)
    pre_out = [nrow_shape] * 3
    h, a, c = pl.pallas_call(
        _pre_body,
        grid=(GRID,),
        in_specs=[
            _rows_spec(HID), pl.BlockSpec((BLK, 1), lambda i: (i, 0)),
            _full_spec((HID, HID)), _full_spec((1, HID)),
            _full_spec((N_BATCH, N_BATCH)), _full_spec((N_BATCH, HID)),
            _full_spec((1, HID)),
            _full_spec((HID, HID)), _full_spec((1, HID)),
            _full_spec((HID, HID)),
        ],
        out_specs=[_rows_spec(HID)] * 3,
        out_shape=pre_out,
    )(x, b2d, nwt, nb, dfp, dwt, db, wat[0], b1[0], wbt[0])

    s2, cnt_full = _sc_edge_cnt(a, c, src, dst, zs)
    c2 = cnt_full[:, :, :8]

    mid_in_specs = [
        _rows_spec(HID), _part_spec(HID), _part_spec(8),
        _full_spec((HID, HID)), _full_spec((1, HID)),
        _full_spec((HID, HID)), _full_spec((1, HID)),
        _full_spec((HID, HID)), _full_spec((HID, HID)), _full_spec((1, HID)),
        _full_spec((1, HID)), _full_spec((1, HID)),
    ]

    for i in range(N_LAYERS - 1):
        h, a, c = pl.pallas_call(
            _mid_body,
            grid=(GRID,),
            in_specs=mid_in_specs + [
                _full_spec((HID, HID)), _full_spec((1, HID)),
                _full_spec((HID, HID)),
            ],
            out_specs=[_rows_spec(HID)] * 3,
            out_shape=pre_out,
        )(h, s2, c2, w2t[i], b2[i], swt[i], sb[i], owat[i], owbt[i], ob[i],
          gs[i], bbs[i], wat[i + 1], b1[i + 1], wbt[i + 1])
        s2 = _sc_edge(a, c, src, dst, zs)

    y = pl.pallas_call(
        _post_body,
        grid=(GRID,),
        in_specs=mid_in_specs + [
            _full_spec((HID, N_OUT)), _full_spec((1, N_OUT)),
        ],
        out_specs=_rows_spec(N_OUT),
        out_shape=jax.ShapeDtypeStruct((N_NODES, N_OUT), F32),
    )(h, s2, c2, w2t[2], b2[2], swt[2], sb[2], owat[2], owbt[2], ob[2],
      gs[2], bbs[2], pjt, pjb)
    return y


# speedup vs baseline: 2.0055x; 1.3041x over previous
import functools

import jax
import jax.numpy as jnp
import numpy as np
from jax import lax
from jax.experimental import pallas as pl
from jax.experimental.pallas import tpu as pltpu
from jax.experimental.pallas import tpu_sc as plsc

N_NODES = 10000
N_EDGES = 320000
HID = 128
N_OUT = 64
N_BATCH = 64
D_DRONE = 51
N_LAYERS = 3
LANES = 16

NC = 2
NS = 16
NW = NC * NS
EPW = N_EDGES // NW
CHUNK = 80
NCHUNK = EPW // CHUNK
CPR = 400
NCP = N_NODES // CPR

BLK = 400
GRID = N_NODES // BLK
F32 = jnp.float32

_SC_MESH = plsc.VectorSubcoreMesh(core_axis_name="c", subcore_axis_name="s")


BF16 = jnp.bfloat16
assert (NCHUNK - 1) % 4 == 0


def _edge_chunk_loop(wid, a_hbm, c_hbm, src_hbm, dst_hbm,
                     srcs, dsts, avs, cvs, s_sh, gsems, ssems, isems):
    def issue_idx(t, s, m):
        off = wid * EPW + t * CHUNK
        pltpu.async_copy(src_hbm.at[pl.ds(off, CHUNK)], srcs[s], isems[m])
        pltpu.async_copy(dst_hbm.at[pl.ds(off, CHUNK)], dsts[s], isems[m])

    def wait_idx(t, s, m):
        off = wid * EPW + t * CHUNK
        pltpu.make_async_copy(src_hbm.at[pl.ds(off, CHUNK)], srcs[s],
                              isems[m]).wait()
        pltpu.make_async_copy(dst_hbm.at[pl.ds(off, CHUNK)], dsts[s],
                              isems[m]).wait()

    def issue_gathers(p, s):
        pltpu.async_copy(a_hbm.at[srcs[s]], avs[p], gsems[p])
        pltpu.async_copy(c_hbm.at[dsts[s]], cvs[p], gsems[p])

    def wait_gathers(p, s):
        pltpu.make_async_copy(a_hbm.at[srcs[s]], avs[p], gsems[p]).wait()
        pltpu.make_async_copy(c_hbm.at[dsts[s]], cvs[p], gsems[p]).wait()

    def issue_scatter(p, s):
        pltpu.async_copy(avs[p], s_sh.at[dsts[s]], ssems[p], add=True)

    def wait_scatter(p, s):
        pltpu.make_async_copy(avs[p], s_sh.at[dsts[s]], ssems[p]).wait()

    def compute(p):
        ap = avs[p]
        cp = cvs[p]

        def row_body(i2, cc):
            for u in range(2):
                i = i2 * 2 + u
                for g in range(HID // LANES):
                    sl = pl.ds(g * LANES, LANES)
                    ap[i, sl] = jnp.maximum(ap[i, sl] - cp[i, sl], 0.0)
            return cc

        lax.fori_loop(0, CHUNK // 2, row_body, 0)

    issue_idx(0, 0, 0)
    wait_idx(0, 0, 0)
    issue_gathers(0, 0)
    issue_idx(1, 1, 1)

    def outer(j, carry):
        for b in range(4):
            t = j * 4 + b
            p = b % 2
            q = 1 - p
            s = b
            s1 = (b + 1) % 4
            s2 = (b + 2) % 4

            @pl.when(t >= 1)
            def _():
                wait_scatter(q, (b + 3) % 4)

            @pl.when(t + 2 < NCHUNK)
            def _():
                issue_idx(t + 2, s2, p)

            wait_idx(t + 1, s1, q)
            issue_gathers(q, s1)
            wait_gathers(p, s)
            compute(p)
            issue_scatter(p, s)
        return carry

    lax.fori_loop(0, (NCHUNK - 1) // 4, outer, 0)
    wait_scatter(1, 3)
    wait_gathers(0, 0)
    compute(0)
    issue_scatter(0, 0)
    wait_scatter(0, 0)


def _rows_copy(sid, pairs):
    r0 = sid * CPR
    for src, dst in pairs:
        pltpu.sync_copy(src(r0), dst(r0))

    @pl.when(sid + NS < NCP)
    def _():
        r1 = (sid + NS) * CPR
        for src, dst in pairs:
            pltpu.sync_copy(src(r1), dst(r1))


_SC_SCRATCH = (
    [pltpu.VMEM((CHUNK,), jnp.int32) for _ in range(8)]
    + [pltpu.VMEM((CHUNK, HID), F32) for _ in range(4)]
    + [pltpu.VMEM_SHARED((N_NODES, HID), F32)]
    + [pltpu.SemaphoreType.DMA for _ in range(6)]
)


def _sc_unpack_bufs(bufs):
    srcs = bufs[0:4]
    dsts = bufs[4:8]
    avs = bufs[8:10]
    cvs = bufs[10:12]
    s_sh = bufs[12]
    gsems = bufs[13:15]
    ssems = bufs[15:17]
    isems = bufs[17:19]
    return srcs, dsts, avs, cvs, s_sh, gsems, ssems, isems


@functools.partial(
    pl.kernel,
    out_type=(
        jax.ShapeDtypeStruct((NC, N_NODES, HID), F32),
        jax.ShapeDtypeStruct((NC, N_NODES, HID), F32),
    ),
    mesh=_SC_MESH,
    scratch_types=_SC_SCRATCH,
)
def _sc_edge_cnt(a_hbm, c_hbm, src_hbm, dst_hbm, zs_hbm,
                 s_out, cnt_out, *bufs):
    srcs, dsts, avs, cvs, s_sh, gsems, ssems, isems = _sc_unpack_bufs(bufs)
    cid = lax.axis_index("c")
    sid = lax.axis_index("s")
    wid = sid * NC + cid
    _rows_copy(sid, [
        (lambda r: zs_hbm.at[pl.ds(r, CPR)], lambda r: s_sh.at[pl.ds(r, CPR)]),
    ])
    plsc.subcore_barrier()
    _edge_chunk_loop(wid, a_hbm, c_hbm, src_hbm, dst_hbm,
                     srcs, dsts, avs, cvs, s_sh, gsems, ssems, isems)
    plsc.subcore_barrier()
    _rows_copy(sid, [
        (lambda r: s_sh.at[pl.ds(r, CPR)],
         lambda r: s_out.at[cid, pl.ds(r, CPR)]),
    ])
    plsc.subcore_barrier()
    _rows_copy(sid, [
        (lambda r: zs_hbm.at[pl.ds(r, CPR)], lambda r: s_sh.at[pl.ds(r, CPR)]),
    ])

    ones_v = cvs[0]

    def ones_row(i, cc):
        for g in range(HID // LANES):
            ones_v[i, pl.ds(g * LANES, LANES)] = jnp.full((LANES,), 1.0, F32)
        return cc

    lax.fori_loop(0, CHUNK, ones_row, 0)
    plsc.subcore_barrier()

    def cnt_load_issue(t, p):
        off = wid * EPW + t * CHUNK
        pltpu.sync_copy(dst_hbm.at[pl.ds(off, CHUNK)], dsts[p])
        pltpu.async_copy(ones_v, s_sh.at[dsts[p]], ssems[p], add=True)

    def cnt_wait(p):
        pltpu.make_async_copy(ones_v, s_sh.at[dsts[p]], ssems[p]).wait()

    def cnt_outer(k, carry):
        for b in range(2):
            t = k * 2 + b

            @pl.when(t >= 2)
            def _():
                cnt_wait(b)

            cnt_load_issue(t, b)
        return carry

    lax.fori_loop(0, (NCHUNK - 1) // 2, cnt_outer, 0)
    cnt_wait(0)
    cnt_load_issue(NCHUNK - 1, 0)
    cnt_wait(0)
    cnt_wait(1)
    plsc.subcore_barrier()
    _rows_copy(sid, [
        (lambda r: s_sh.at[pl.ds(r, CPR)],
         lambda r: cnt_out.at[cid, pl.ds(r, CPR)]),
    ])


@functools.partial(
    pl.kernel,
    out_type=jax.ShapeDtypeStruct((NC, N_NODES, HID), F32),
    mesh=_SC_MESH,
    scratch_types=_SC_SCRATCH,
)
def _sc_edge(a_hbm, c_hbm, src_hbm, dst_hbm, zs_hbm,
             s_out, *bufs):
    srcs, dsts, avs, cvs, s_sh, gsems, ssems, isems = _sc_unpack_bufs(bufs)
    cid = lax.axis_index("c")
    sid = lax.axis_index("s")
    wid = sid * NC + cid
    _rows_copy(sid, [
        (lambda r: zs_hbm.at[pl.ds(r, CPR)], lambda r: s_sh.at[pl.ds(r, CPR)]),
    ])
    plsc.subcore_barrier()
    _edge_chunk_loop(wid, a_hbm, c_hbm, src_hbm, dst_hbm,
                     srcs, dsts, avs, cvs, s_sh, gsems, ssems, isems)
    plsc.subcore_barrier()
    _rows_copy(sid, [
        (lambda r: s_sh.at[pl.ds(r, CPR)],
         lambda r: s_out.at[cid, pl.ds(r, CPR)]),
    ])


def _dot(a, b):
    return jnp.dot(a, b, preferred_element_type=F32)


def _pre_body(x, b2d, nwt, nb, dfp, dwt, db, wat, b1, wbt,
              h_out, a_out, c_out):
    demb = _dot(dfp[...], dwt[...]) + db[...]
    h = _dot(x[...], nwt[...]) + nb[...]
    oneh = (b2d[...] == lax.broadcasted_iota(jnp.int32, (BLK, N_BATCH), 1)
            ).astype(F32)
    h = h + _dot(oneh, demb)
    h_out[...] = h
    a_out[...] = _dot(h, wat[...]) + b1[...]
    c_out[...] = _dot(h, wbt[...])


def _layer_update(h, s2, c2, w2t, b2, swt, sb, owat, owbt, ob, g, bb):
    sarr = s2[...]
    s = sarr[0] + sarr[1]
    carr = c2[...]
    cnt = carr[0, :, 0:1] + carr[1, :, 0:1]
    aggr = (_dot(s, w2t[...]) + cnt * b2[...]) / jnp.maximum(cnt, 1.0)
    selfp = _dot(h[...], swt[...]) + sb[...]
    cc = _dot(selfp, owat[...]) + _dot(aggr, owbt[...]) + ob[...]
    mu = jnp.mean(cc, axis=-1, keepdims=True)
    var = jnp.mean((cc - mu) ** 2, axis=-1, keepdims=True)
    ln = (cc - mu) * lax.rsqrt(var + 1e-5) * g[...] + bb[...]
    return h[...] + jnp.maximum(ln, 0.0)


def _mid_body(h, s2, c2, w2t, b2, swt, sb, owat, owbt, ob, g, bb,
              want, b1n, wbnt, h_out, a_out, c_out):
    hn = _layer_update(h, s2, c2, w2t, b2, swt, sb, owat, owbt, ob, g, bb)
    h_out[...] = hn
    a_out[...] = _dot(hn, want[...]) + b1n[...]
    c_out[...] = _dot(hn, wbnt[...])


def _post_body(h, s2, c2, w2t, b2, swt, sb, owat, owbt, ob, g, bb,
               pjt, pjb, y_out):
    hn = _layer_update(h, s2, c2, w2t, b2, swt, sb, owat, owbt, ob, g, bb)
    y_out[...] = _dot(hn, pjt[...]) + pjb[...]


def _rows_spec(ncol):
    return pl.BlockSpec((BLK, ncol), lambda i: (i, 0))


def _full_spec(shape):
    nd = len(shape)
    return pl.BlockSpec(shape, lambda i, _nd=nd: (0,) * _nd)


def _part_spec(ncol):
    return pl.BlockSpec((NC, BLK, ncol), lambda i: (0, i, 0))


def kernel(x, edge_index, drone_feat, batch, node_w, node_b, drone_w, drone_b,
           edge_w1, edge_b1, edge_w2, edge_b2, self_w, self_b, out_w, out_b,
           ln_g, ln_b, proj_w, proj_b):
    src = edge_index[0]
    dst = edge_index[1]
    b2d = batch.reshape(N_NODES, 1)

    nwt = node_w.T
    nb = node_b.reshape(1, HID)
    dfp = jnp.pad(drone_feat, ((0, 0), (0, N_BATCH - D_DRONE)))
    dwt = jnp.pad(drone_w.T, ((0, N_BATCH - D_DRONE), (0, 0)))
    db = drone_b.reshape(1, HID)

    wat = [(edge_w1[i, :, :HID] + edge_w1[i, :, HID:]).T for i in range(N_LAYERS)]
    wbt = [edge_w1[i, :, HID:].T for i in range(N_LAYERS)]
    b1 = [edge_b1[i].reshape(1, HID) for i in range(N_LAYERS)]
    w2t = [edge_w2[i].T for i in range(N_LAYERS)]
    b2 = [edge_b2[i].reshape(1, HID) for i in range(N_LAYERS)]
    swt = [self_w[i].T for i in range(N_LAYERS)]
    sb = [self_b[i].reshape(1, HID) for i in range(N_LAYERS)]
    owat = [out_w[i, :, :HID].T for i in range(N_LAYERS)]
    owbt = [out_w[i, :, HID:].T for i in range(N_LAYERS)]
    ob = [out_b[i].reshape(1, HID) for i in range(N_LAYERS)]
    gs = [ln_g[i].reshape(1, HID) for i in range(N_LAYERS)]
    bbs = [ln_b[i].reshape(1, HID) for i in range(N_LAYERS)]
    pjt = proj_w.T
    pjb = proj_b.reshape(1, N_OUT)

    zs = jnp.zeros((N_NODES, HID), F32)

    nrow_shape = jax.ShapeDtypeStruct((N_NODES, HID), F32)
    pre_out = [nrow_shape] * 3
    h, a, c = pl.pallas_call(
        _pre_body,
        grid=(GRID,),
        in_specs=[
            _rows_spec(HID), pl.BlockSpec((BLK, 1), lambda i: (i, 0)),
            _full_spec((HID, HID)), _full_spec((1, HID)),
            _full_spec((N_BATCH, N_BATCH)), _full_spec((N_BATCH, HID)),
            _full_spec((1, HID)),
            _full_spec((HID, HID)), _full_spec((1, HID)),
            _full_spec((HID, HID)),
        ],
        out_specs=[_rows_spec(HID)] * 3,
        out_shape=pre_out,
    )(x, b2d, nwt, nb, dfp, dwt, db, wat[0], b1[0], wbt[0])

    s2, cnt_full = _sc_edge_cnt(a, c, src, dst, zs)
    c2 = cnt_full[:, :, :8]

    mid_in_specs = [
        _rows_spec(HID), _part_spec(HID), _part_spec(8),
        _full_spec((HID, HID)), _full_spec((1, HID)),
        _full_spec((HID, HID)), _full_spec((1, HID)),
        _full_spec((HID, HID)), _full_spec((HID, HID)), _full_spec((1, HID)),
        _full_spec((1, HID)), _full_spec((1, HID)),
    ]

    for i in range(N_LAYERS - 1):
        h, a, c = pl.pallas_call(
            _mid_body,
            grid=(GRID,),
            in_specs=mid_in_specs + [
                _full_spec((HID, HID)), _full_spec((1, HID)),
                _full_spec((HID, HID)),
            ],
            out_specs=[_rows_spec(HID)] * 3,
            out_shape=pre_out,
        )(h, s2, c2, w2t[i], b2[i], swt[i], sb[i], owat[i], owbt[i], ob[i],
          gs[i], bbs[i], wat[i + 1], b1[i + 1], wbt[i + 1])
        s2 = _sc_edge(a, c, src, dst, zs)

    y = pl.pallas_call(
        _post_body,
        grid=(GRID,),
        in_specs=mid_in_specs + [
            _full_spec((HID, N_OUT)), _full_spec((1, N_OUT)),
        ],
        out_specs=_rows_spec(N_OUT),
        out_shape=jax.ShapeDtypeStruct((N_NODES, N_OUT), F32),
    )(h, s2, c2, w2t[2], b2[2], swt[2], sb[2], owat[2], owbt[2], ob[2],
      gs[2], bbs[2], pjt, pjb)
    return y
